# Initial kernel scaffold; baseline (speedup 1.0000x reference)
import jax
import jax.numpy as jnp
from jax.experimental import pallas as pl


def kernel(x_nchw, gamma, beta, w_point, w_depth, b_depth):
    raise NotImplementedError("write your pallas kernel here")


# speedup vs baseline: 1.5649x; 1.5649x over previous
import functools

import jax
import jax.numpy as jnp
from jax.experimental import pallas as pl
from jax.experimental.pallas import tpu as pltpu


def _stats_kernel(x_ref, o_ref):
    x = x_ref[0].astype(jnp.float32)
    o_ref[0, :, 0:1] = jnp.sum(x, axis=1, keepdims=True)
    o_ref[0, :, 1:2] = jnp.sum(x * x, axis=1, keepdims=True)


def _conv_kernel(x_ref, wp_ref, bp_ref, wd_ref, bd_ref, o_ref,
                 *, H, W, neg_slope):
    HW = H * W
    x = x_ref[0].astype(jnp.float32)

    y = jnp.dot(wp_ref[...], x, preferred_element_type=jnp.float32)
    y = y + bp_ref[...]
    y = jnp.maximum(y, neg_slope * y)

    lane = jax.lax.broadcasted_iota(jnp.int32, (1, HW), 1)
    ww = lane % W
    hh = lane // W
    left = (ww >= 1).astype(jnp.float32)
    right = (ww < W - 1).astype(jnp.float32)
    top = (hh >= 1).astype(jnp.float32)
    bot = (hh < H - 1).astype(jnp.float32)

    wd = wd_ref[...].astype(jnp.float32)
    bd = bd_ref[...].astype(jnp.float32)

    ym = pltpu.roll(y, shift=1, axis=1) * left
    yp = pltpu.roll(y, shift=HW - 1, axis=1) * right

    def row(k):
        return ym * wd[:, k:k + 1] + y * wd[:, k + 1:k + 2] \
            + yp * wd[:, k + 2:k + 3]

    acc = row(3) + bd
    acc = acc + pltpu.roll(row(0), shift=W, axis=1) * top
    acc = acc + pltpu.roll(row(6), shift=HW - W, axis=1) * bot
    acc = jnp.maximum(acc, neg_slope * acc)
    o_ref[0] = acc.astype(o_ref.dtype)


def kernel(x_nchw, gamma, beta, w_point, w_depth, b_depth):
    eps = 1e-5
    neg_slope = 0.01
    N, Cin, H, W = x_nchw.shape
    Cout = w_point.shape[1]
    HW = H * W

    x = x_nchw.reshape(N, Cin, HW)

    stats = pl.pallas_call(
        _stats_kernel,
        out_shape=jax.ShapeDtypeStruct((N, Cin, 2), jnp.float32),
        grid=(N,),
        in_specs=[pl.BlockSpec((1, Cin, HW), lambda n: (n, 0, 0))],
        out_specs=pl.BlockSpec((1, Cin, 2), lambda n: (n, 0, 0)),
        compiler_params=pltpu.CompilerParams(
            dimension_semantics=("parallel",),
            vmem_limit_bytes=60 * 1024 * 1024),
    )(x)

    tot = jnp.sum(stats, axis=0)
    cnt = jnp.float32(N * HW)
    mean = tot[:, 0] / cnt
    var = jnp.maximum(tot[:, 1] / cnt - mean * mean, 0.0)
    scale = gamma * jax.lax.rsqrt(var + eps)
    shift = beta - mean * scale
    wp_fused = (w_point * scale[:, None]).T
    bp_fused = (shift @ w_point).reshape(Cout, 1)
    wd = jnp.transpose(w_depth.reshape(9, Cout))
    bd = b_depth.reshape(Cout, 1)

    out = pl.pallas_call(
        functools.partial(_conv_kernel, H=H, W=W, neg_slope=neg_slope),
        out_shape=jax.ShapeDtypeStruct((N, Cout, HW), x_nchw.dtype),
        grid=(N,),
        in_specs=[
            pl.BlockSpec((1, Cin, HW), lambda n: (n, 0, 0)),
            pl.BlockSpec((Cout, Cin), lambda n: (0, 0)),
            pl.BlockSpec((Cout, 1), lambda n: (0, 0)),
            pl.BlockSpec((Cout, 9), lambda n: (0, 0)),
            pl.BlockSpec((Cout, 1), lambda n: (0, 0)),
        ],
        out_specs=pl.BlockSpec((1, Cout, HW), lambda n: (n, 0, 0)),
        compiler_params=pltpu.CompilerParams(
            dimension_semantics=("parallel",),
            vmem_limit_bytes=60 * 1024 * 1024),
    )(x, wp_fused, bp_fused, wd, bd)

    return out.reshape(N, Cout, H, W)
